# baseline (speedup 1.0000x reference)
import functools

import jax
import jax.numpy as jnp
from jax import lax
from jax.experimental import pallas as pl
from jax.experimental.pallas import tpu as pltpu
from jax.experimental.pallas import tpu_sc as plsc

B = 4096
DU = 768
DA = 256
HEADS = 8
H_SC = 5
H_TC = HEADS - H_SC
NW = 32
ROWS_W = B // NW
CH = 32
NCH = ROWS_W // CH
REP = 16
BB = 512


def _sc_body(xt_hbm, rew_hbm, sub_hbm, act_hbm, out_hbm,
             xtbuf, actrep, act_v, mbuf8, rew_v, sub_v,
             in_sem, out_sem, setup_sem):
    wid = lax.axis_index("c") * 16 + lax.axis_index("s")
    base = wid * ROWS_W

    def stage(c, slot):
        row0 = base + c * CH
        return pltpu.async_copy(
            xt_hbm.at[pl.ds(row0, CH), 0, :], xtbuf.at[slot], in_sem)

    first = stage(0, 0)

    c_rew = pltpu.async_copy(rew_hbm.at[pl.ds(base, ROWS_W)], rew_v, setup_sem)
    c_sub = pltpu.async_copy(sub_hbm.at[pl.ds(base, ROWS_W), :], sub_v,
                             setup_sem)
    c_act = pltpu.async_copy(act_hbm, act_v, setup_sem)
    c_rew.wait()
    c_sub.wait()
    c_act.wait()

    def rep_body(i, _):
        h = lax.div(i, DA // 16)
        v = lax.rem(i, DA // 16)
        x = act_v[h, pl.ds(v * 16, 16)]

        def rep_inner(r, _):
            actrep[h, r, pl.ds(v * 16, 16)] = x
            return 0
        lax.fori_loop(0, REP, rep_inner, 0)
        return 0
    lax.fori_loop(0, H_SC * (DA // 16), rep_body, 0)

    def fire(c, slot):
        row0 = base + c * CH
        handles = []
        for h in range(H_SC):
            handles.append(pltpu.async_copy(
                xtbuf.at[slot],
                out_hbm.at[h, pl.ds(row0, CH), 0, pl.ds(0, DU)], out_sem))
            for q in range(CH // REP):
                handles.append(pltpu.async_copy(
                    actrep.at[h],
                    out_hbm.at[h, pl.ds(row0 + q * REP, REP), 0,
                               pl.ds(DU, DA)], out_sem))
        return handles

    pending = [None, None]
    first.wait()
    pending[0] = fire(0, 0)
    for c in range(1, NCH):
        slot = c % 2
        if pending[slot] is not None:
            for hnd in pending[slot]:
                hnd.wait()
            pending[slot] = None
        stage(c, slot).wait()
        pending[slot] = fire(c, slot)
    for p in pending:
        if p is not None:
            for hnd in p:
                hnd.wait()

    lanes = lax.iota(jnp.int32, 16)

    def count_body(j, acc):
        ridx = j * 16 + lanes
        rok = plsc.load_gather(rew_v, [ridx]) == 1
        for h in range(H_SC):
            sub16 = plsc.load_gather(
                sub_v, [ridx, jnp.full((16,), 0, jnp.int32) + h])
            acc = acc + jnp.where(rok & (sub16 >= 0.1), 1.0, 0.0)
        return acc
    total = jnp.sum(lax.fori_loop(
        0, ROWS_W // 16, count_body, jnp.zeros((16,), jnp.float32)))

    def sweep(i, _):
        g = lax.div(i, H_SC)
        h = lax.rem(i, H_SC)
        off = g * 8
        row0 = base + off
        ridx = off + lax.rem(lanes, 8)
        rew16 = plsc.load_gather(rew_v, [ridx])
        sub16 = plsc.load_gather(sub_v, [ridx, jnp.full((16,), 0, jnp.int32) + h])
        mf = jnp.where((rew16 == 1) & (sub16 >= 0.1), 1.0, 0.0)
        nsel = jnp.sum(jnp.where(lanes < 8, mf, 0.0))

        @pl.when(nsel < 7.5)
        def _fix():
            pltpu.sync_copy(xt_hbm.at[pl.ds(row0, 8), 0, :], mbuf8)

            def rowfn(r, _):
                mr = jnp.max(jnp.where(lanes == r, mf, 0.0))

                def vecfn(v, _):
                    sl = pl.ds(v * 16, 16)
                    mbuf8[r, sl] = mbuf8[r, sl] * mr
                    return 0
                lax.fori_loop(0, DU // 16, vecfn, 0)
                return 0
            lax.fori_loop(0, 8, rowfn, 0)
            pltpu.sync_copy(
                mbuf8, out_hbm.at[h, pl.ds(row0, 8), 0, pl.ds(0, DU)])
        return 0

    @pl.when(total < ROWS_W * H_SC - 0.5)
    def _full_sweep():
        lax.fori_loop(0, (ROWS_W // 8) * H_SC, sweep, 0)


_sc_call = functools.partial(
    pl.kernel,
    out_type=jax.ShapeDtypeStruct((H_SC, B, 1, DU + DA), jnp.float32),
    mesh=plsc.VectorSubcoreMesh(core_axis_name="c", subcore_axis_name="s"),
    compiler_params=pltpu.CompilerParams(needs_layout_passes=False),
    scratch_types=[
        pltpu.VMEM((2, CH, DU), jnp.float32),
        pltpu.VMEM((H_SC, REP, DA), jnp.float32),
        pltpu.VMEM((HEADS, DA), jnp.float32),
        pltpu.VMEM((8, DU), jnp.float32),
        pltpu.VMEM((ROWS_W,), jnp.int32),
        pltpu.VMEM((ROWS_W, HEADS), jnp.float32),
        pltpu.SemaphoreType.DMA,
        pltpu.SemaphoreType.DMA,
        pltpu.SemaphoreType.DMA,
    ],
)(_sc_body)


def _tc_body(xt_ref, rew_ref, sub_ref, act_ref, out_ref):
    m = (rew_ref[:, 0:1] == 1) & (sub_ref[0, :, :] >= 0.1)
    sel = jnp.where(m, xt_ref[:, 0, :], 0.0)
    act = jnp.broadcast_to(act_ref[0, :, :], (BB, DA))
    out_ref[0, :, 0, :] = jnp.concatenate([sel, act], axis=1)


def _tc_call(xt, rew2, sub3, act3):
    return pl.pallas_call(
        _tc_body,
        grid=(B // BB, H_TC),
        in_specs=[
            pl.BlockSpec((BB, 1, DU), lambda i, h: (i, 0, 0)),
            pl.BlockSpec((BB, 1), lambda i, h: (i, 0)),
            pl.BlockSpec((1, BB, 1), lambda i, h: (h, i, 0)),
            pl.BlockSpec((1, 1, DA), lambda i, h: (h, 0, 0)),
        ],
        out_specs=pl.BlockSpec((1, BB, 1, DU + DA), lambda i, h: (h, i, 0, 0)),
        out_shape=jax.ShapeDtypeStruct((H_TC, B, 1, DU + DA), jnp.float32),
    )(xt, rew2, sub3, act3)


def kernel(xt, rewards, subset, action):
    sc_out = _sc_call(xt, rewards, subset, action)
    rew2 = rewards.reshape(B, 1)
    sub3 = subset.T[H_SC:].reshape(H_TC, B, 1)
    act3 = action[H_SC:].reshape(H_TC, 1, DA)
    tc_out = _tc_call(xt, rew2, sub3, act3)
    return jnp.concatenate([sc_out, tc_out], axis=0)


# speedup vs baseline: 2.7409x; 2.7409x over previous
import functools

import jax
import jax.numpy as jnp
from jax import lax
from jax.experimental import pallas as pl
from jax.experimental.pallas import tpu as pltpu
from jax.experimental.pallas import tpu_sc as plsc

B = 4096
DU = 768
DA = 256
HEADS = 8
NW = 32
ROWS_W = B // NW
CH = 32
NCH = ROWS_W // CH
REP = 16


def _sc_body(xt_hbm, rew_hbm, sub_hbm, act_hbm, out_hbm,
             xtbuf, actrep, act_v, mbuf8, rew_v, sub_v,
             in_sem, out_sem, setup_sem):
    wid = lax.axis_index("c") * 16 + lax.axis_index("s")
    base = wid * ROWS_W

    def stage(c, slot):
        row0 = base + c * CH
        return pltpu.async_copy(
            xt_hbm.at[pl.ds(row0, CH), 0, :], xtbuf.at[slot], in_sem)

    first = stage(0, 0)

    c_rew = pltpu.async_copy(rew_hbm.at[pl.ds(base, ROWS_W)], rew_v, setup_sem)
    c_sub = pltpu.async_copy(sub_hbm.at[pl.ds(base, ROWS_W), :], sub_v,
                             setup_sem)
    c_act = pltpu.async_copy(act_hbm, act_v, setup_sem)
    c_rew.wait()
    c_sub.wait()
    c_act.wait()

    def rep_body(i, _):
        h = lax.div(i, DA // 16)
        v = lax.rem(i, DA // 16)
        x = act_v[h, pl.ds(v * 16, 16)]

        def rep_inner(r, _):
            actrep[h, r, pl.ds(v * 16, 16)] = x
            return 0
        lax.fori_loop(0, REP, rep_inner, 0)
        return 0
    lax.fori_loop(0, HEADS * (DA // 16), rep_body, 0)

    def fire(c, slot):
        row0 = base + c * CH
        handles = []
        for h in range(HEADS):
            handles.append(pltpu.async_copy(
                xtbuf.at[slot],
                out_hbm.at[h, pl.ds(row0, CH), 0, pl.ds(0, DU)], out_sem))
            for q in range(CH // REP):
                handles.append(pltpu.async_copy(
                    actrep.at[h],
                    out_hbm.at[h, pl.ds(row0 + q * REP, REP), 0,
                               pl.ds(DU, DA)], out_sem))
        return handles

    pending = [None, None]
    first.wait()
    pending[0] = fire(0, 0)
    for c in range(1, NCH):
        slot = c % 2
        if pending[slot] is not None:
            for hnd in pending[slot]:
                hnd.wait()
            pending[slot] = None
        stage(c, slot).wait()
        pending[slot] = fire(c, slot)
    for p in pending:
        if p is not None:
            for hnd in p:
                hnd.wait()

    lanes = lax.iota(jnp.int32, 16)

    def count_body(j, acc):
        ridx = j * 16 + lanes
        rok = plsc.load_gather(rew_v, [ridx]) == 1
        for h in range(HEADS):
            sub16 = plsc.load_gather(
                sub_v, [ridx, jnp.full((16,), 0, jnp.int32) + h])
            acc = acc + jnp.where(rok & (sub16 >= 0.1), 1.0, 0.0)
        return acc
    total = jnp.sum(lax.fori_loop(
        0, ROWS_W // 16, count_body, jnp.zeros((16,), jnp.float32)))

    def sweep(i, _):
        g = lax.div(i, HEADS)
        h = lax.rem(i, HEADS)
        off = g * 8
        row0 = base + off
        ridx = off + lax.rem(lanes, 8)
        rew16 = plsc.load_gather(rew_v, [ridx])
        sub16 = plsc.load_gather(sub_v, [ridx, jnp.full((16,), 0, jnp.int32) + h])
        mf = jnp.where((rew16 == 1) & (sub16 >= 0.1), 1.0, 0.0)
        nsel = jnp.sum(jnp.where(lanes < 8, mf, 0.0))

        @pl.when(nsel < 7.5)
        def _fix():
            pltpu.sync_copy(xt_hbm.at[pl.ds(row0, 8), 0, :], mbuf8)

            def rowfn(r, _):
                mr = jnp.max(jnp.where(lanes == r, mf, 0.0))

                def vecfn(v, _):
                    sl = pl.ds(v * 16, 16)
                    mbuf8[r, sl] = mbuf8[r, sl] * mr
                    return 0
                lax.fori_loop(0, DU // 16, vecfn, 0)
                return 0
            lax.fori_loop(0, 8, rowfn, 0)
            pltpu.sync_copy(
                mbuf8, out_hbm.at[h, pl.ds(row0, 8), 0, pl.ds(0, DU)])
        return 0

    @pl.when(total < ROWS_W * HEADS - 0.5)
    def _full_sweep():
        lax.fori_loop(0, (ROWS_W // 8) * HEADS, sweep, 0)


_sc_call = functools.partial(
    pl.kernel,
    out_type=jax.ShapeDtypeStruct((HEADS, B, 1, DU + DA), jnp.float32),
    mesh=plsc.VectorSubcoreMesh(core_axis_name="c", subcore_axis_name="s"),
    compiler_params=pltpu.CompilerParams(needs_layout_passes=False),
    scratch_types=[
        pltpu.VMEM((2, CH, DU), jnp.float32),
        pltpu.VMEM((HEADS, REP, DA), jnp.float32),
        pltpu.VMEM((HEADS, DA), jnp.float32),
        pltpu.VMEM((8, DU), jnp.float32),
        pltpu.VMEM((ROWS_W,), jnp.int32),
        pltpu.VMEM((ROWS_W, HEADS), jnp.float32),
        pltpu.SemaphoreType.DMA,
        pltpu.SemaphoreType.DMA,
        pltpu.SemaphoreType.DMA,
    ],
)(_sc_body)


def kernel(xt, rewards, subset, action):
    return _sc_call(xt, rewards, subset, action)
